# Initial kernel scaffold; baseline (speedup 1.0000x reference)
import jax
import jax.numpy as jnp
from jax.experimental import pallas as pl


def kernel(boxes, scores):
    raise NotImplementedError("write your pallas kernel here")


# speedup vs baseline: 59.0170x; 59.0170x over previous
import functools

import jax
import jax.numpy as jnp
from jax import lax
from jax.experimental import pallas as pl
from jax.experimental.pallas import tpu as pltpu

_TH = 0.7
_N = 5000
_B = 128
_NB = 40
_NP = _NB * _B
_P = 2048
_TOPN = 2000


def _nms_body(b_ref, x1_ref, y1_ref, x2_ref, y2_ref, out_ref, keep_ref):
    f32 = jnp.float32

    keep_ref[:] = jnp.ones((_NB, 1, _B), f32)

    tri = (
        lax.broadcasted_iota(jnp.int32, (_B, _B), 1)
        > lax.broadcasted_iota(jnp.int32, (_B, _B), 0)
    ).astype(f32)

    def blk_body(blk, _):
        pb = b_ref[pl.ds(blk * _B, _B), :]
        px1 = pb[:, 0:1]
        py1 = pb[:, 1:2]
        px2 = pb[:, 2:3]
        py2 = pb[:, 3:4]
        pa = (px2 - px1) * (py2 - py1)

        def iou_mask(c):
            tx1 = x1_ref[c]
            ty1 = y1_ref[c]
            tx2 = x2_ref[c]
            ty2 = y2_ref[c]
            ta = (tx2 - tx1) * (ty2 - ty1)
            xx1 = jnp.maximum(px1, tx1)
            yy1 = jnp.maximum(py1, ty1)
            xx2 = jnp.minimum(px2, tx2)
            yy2 = jnp.minimum(py2, ty2)
            w = jnp.maximum(xx2 - xx1, 0.0)
            h = jnp.maximum(yy2 - yy1, 0.0)
            inter = w * h
            iou = inter / (pa + ta - inter + 1e-8)
            return (iou > _TH).astype(f32)

        m = iou_mask(blk) * tri
        elig = keep_ref[blk]

        def fp_cond(carry):
            return carry[1]

        def fp_body(carry):
            k, _ = carry
            sup = lax.dot_general(
                k, m, (((1,), (0,)), ((), ())), preferred_element_type=f32
            )
            knew = jnp.where(sup > 0.0, 0.0, elig)
            return knew, jnp.any(knew != k)

        k, _ = lax.while_loop(fp_cond, fp_body, (elig, jnp.array(True)))
        keep_ref[blk] = k

        def sup_body(c, _):
            m2 = iou_mask(c)
            sup = lax.dot_general(
                k, m2, (((1,), (0,)), ((), ())), preferred_element_type=f32
            )
            keep_ref[c] = jnp.where(sup > 0.0, 0.0, keep_ref[c])
            return 0

        lax.fori_loop(blk + 1, _NB, sup_body, 0)
        return 0

    lax.fori_loop(0, _NB, blk_body, 0)

    keep = keep_ref[:].reshape(_NB, _B)
    gidx = (
        lax.broadcasted_iota(jnp.int32, (_NB, _B), 0) * _B
        + lax.broadcasted_iota(jnp.int32, (_NB, _B), 1)
    )
    validf = (gidx < _N).astype(f32)
    kv = keep * validf
    nv = (1.0 - keep) * validf

    upper = (
        lax.broadcasted_iota(jnp.int32, (_B, _B), 0)
        <= lax.broadcasted_iota(jnp.int32, (_B, _B), 1)
    ).astype(f32)
    dot = functools.partial(
        lax.dot_general,
        dimension_numbers=(((1,), (0,)), ((), ())),
        preferred_element_type=f32,
    )
    kc = dot(kv, upper)
    nc = dot(nv, upper)
    rsk = kc[:, _B - 1 : _B]
    rsn = nc[:, _B - 1 : _B]
    lstrict = (
        lax.broadcasted_iota(jnp.int32, (_NB, _NB), 1)
        < lax.broadcasted_iota(jnp.int32, (_NB, _NB), 0)
    ).astype(f32)
    offk = dot(lstrict, rsk)
    offn = dot(lstrict, rsn)
    nk = jnp.sum(kv)
    posk = kc - 1.0 + offk
    posn = nc - 1.0 + offn + nk
    pos = jnp.where(kv > 0.0, posk, jnp.where(nv > 0.0, posn, 99999.0))

    out_ref[:] = jnp.zeros((_P, 4), f32)
    piota = lax.broadcasted_iota(jnp.int32, (_P, 1), 0).astype(f32)
    for cb in range(_NB):
        prow = lax.slice(pos, (cb, 0), (cb + 1, _B))
        oh = (piota == prow).astype(f32)
        bb = b_ref[cb * _B : (cb + 1) * _B, :]
        out_ref[:] = out_ref[:] + dot(oh, bb)


def kernel(boxes, scores):
    order = jnp.argsort(-scores)
    bs = jnp.take(boxes, order, axis=0)
    bp = jnp.pad(bs, ((0, _NP - _N), (0, 0)))

    x1 = bp[:, 0].reshape(_NB, 1, _B)
    y1 = bp[:, 1].reshape(_NB, 1, _B)
    x2 = bp[:, 2].reshape(_NB, 1, _B)
    y2 = bp[:, 3].reshape(_NB, 1, _B)

    sel = pl.pallas_call(
        _nms_body,
        out_shape=jax.ShapeDtypeStruct((_P, 4), jnp.float32),
        scratch_shapes=[pltpu.VMEM((_NB, 1, _B), jnp.float32)],
    )(bp, x1, y1, x2, y2)

    batch_col = jnp.zeros((_TOPN, 1), jnp.float32)
    return jnp.concatenate([batch_col, sel[:_TOPN]], axis=1)


# speedup vs baseline: 88.2676x; 1.4956x over previous
import functools

import jax
import jax.numpy as jnp
from jax import lax
from jax.experimental import pallas as pl
from jax.experimental.pallas import tpu as pltpu

_TH = 0.7
_N = 5000
_B = 128
_NB = 40
_NP = _NB * _B
_P = 2048
_TOPN = 2000


def _nms_body(
    b_ref, x1_ref, y1_ref, x2_ref, y2_ref, xw_ref, yw_ref, xW_ref, yW_ref,
    out_ref, keep_ref,
):
    f32 = jnp.float32

    keep_ref[:] = jnp.ones((_NB, 1, _B), f32)

    tri = (
        lax.broadcasted_iota(jnp.int32, (_B, _B), 1)
        > lax.broadcasted_iota(jnp.int32, (_B, _B), 0)
    ).astype(f32)
    colidx = lax.broadcasted_iota(jnp.int32, (1, _NP), 1)

    tx1w = xw_ref[:]
    ty1w = yw_ref[:]
    tx2w = xW_ref[:]
    ty2w = yW_ref[:]
    taw = (tx2w - tx1w) * (ty2w - ty1w)

    def blk_body(blk, _):
        pb = b_ref[pl.ds(blk * _B, _B), :]
        px1 = pb[:, 0:1]
        py1 = pb[:, 1:2]
        px2 = pb[:, 2:3]
        py2 = pb[:, 3:4]
        pa = (px2 - px1) * (py2 - py1)

        tx1 = x1_ref[blk]
        ty1 = y1_ref[blk]
        tx2 = x2_ref[blk]
        ty2 = y2_ref[blk]
        ta = (tx2 - tx1) * (ty2 - ty1)
        xx1 = jnp.maximum(px1, tx1)
        yy1 = jnp.maximum(py1, ty1)
        xx2 = jnp.minimum(px2, tx2)
        yy2 = jnp.minimum(py2, ty2)
        w = jnp.maximum(xx2 - xx1, 0.0)
        h = jnp.maximum(yy2 - yy1, 0.0)
        inter = w * h
        iou = inter / (pa + ta - inter + 1e-8)
        m = (iou > _TH).astype(f32) * tri
        elig = keep_ref[blk]

        def fp_cond(carry):
            return carry[1]

        def fp_body(carry):
            k, _ = carry
            sup = lax.dot_general(
                k, m, (((1,), (0,)), ((), ())), preferred_element_type=f32
            )
            knew = jnp.where(sup > 0.0, 0.0, elig)
            return knew, jnp.any(knew != k)

        k, _ = lax.while_loop(fp_cond, fp_body, (elig, jnp.array(True)))
        keep_ref[blk] = k

        wxx1 = jnp.maximum(px1, tx1w)
        wyy1 = jnp.maximum(py1, ty1w)
        wxx2 = jnp.minimum(px2, tx2w)
        wyy2 = jnp.minimum(py2, ty2w)
        ww = jnp.maximum(wxx2 - wxx1, 0.0)
        wh = jnp.maximum(wyy2 - wyy1, 0.0)
        winter = ww * wh
        wiou = winter / (pa + taw - winter + 1e-8)
        mw = (wiou > _TH).astype(f32)
        sup = lax.dot_general(
            k, mw, (((1,), (0,)), ((), ())), preferred_element_type=f32
        )
        supm = (sup > 0.0) & (colidx >= (blk + 1) * _B)
        for c in range(_NB):
            sc = lax.slice(supm, (0, c * _B), (1, (c + 1) * _B))
            keep_ref[c] = jnp.where(sc, 0.0, keep_ref[c])
        return 0

    lax.fori_loop(0, _NB, blk_body, 0)

    keep = keep_ref[:].reshape(_NB, _B)
    gidx = (
        lax.broadcasted_iota(jnp.int32, (_NB, _B), 0) * _B
        + lax.broadcasted_iota(jnp.int32, (_NB, _B), 1)
    )
    validf = (gidx < _N).astype(f32)
    kv = keep * validf
    nv = (1.0 - keep) * validf

    upper = (
        lax.broadcasted_iota(jnp.int32, (_B, _B), 0)
        <= lax.broadcasted_iota(jnp.int32, (_B, _B), 1)
    ).astype(f32)
    dot = functools.partial(
        lax.dot_general,
        dimension_numbers=(((1,), (0,)), ((), ())),
        preferred_element_type=f32,
    )
    kc = dot(kv, upper)
    nc = dot(nv, upper)
    rsk = kc[:, _B - 1 : _B]
    rsn = nc[:, _B - 1 : _B]
    lstrict = (
        lax.broadcasted_iota(jnp.int32, (_NB, _NB), 1)
        < lax.broadcasted_iota(jnp.int32, (_NB, _NB), 0)
    ).astype(f32)
    offk = dot(lstrict, rsk)
    offn = dot(lstrict, rsn)
    nk = jnp.sum(kv)
    posk = kc - 1.0 + offk
    posn = nc - 1.0 + offn + nk
    pos = jnp.where(kv > 0.0, posk, jnp.where(nv > 0.0, posn, 99999.0))

    out_ref[:] = jnp.zeros((_P, 4), f32)
    piota = lax.broadcasted_iota(jnp.int32, (_P, 1), 0).astype(f32)
    for cb in range(_NB):
        prow = lax.slice(pos, (cb, 0), (cb + 1, _B))
        oh = (piota == prow).astype(f32)
        bb = b_ref[cb * _B : (cb + 1) * _B, :]
        out_ref[:] = out_ref[:] + dot(oh, bb)


def kernel(boxes, scores):
    order = jnp.argsort(-scores)
    bs = jnp.take(boxes, order, axis=0)
    bp = jnp.pad(bs, ((0, _NP - _N), (0, 0)))

    x1 = bp[:, 0].reshape(_NB, 1, _B)
    y1 = bp[:, 1].reshape(_NB, 1, _B)
    x2 = bp[:, 2].reshape(_NB, 1, _B)
    y2 = bp[:, 3].reshape(_NB, 1, _B)
    xw = bp[:, 0].reshape(1, _NP)
    yw = bp[:, 1].reshape(1, _NP)
    xW = bp[:, 2].reshape(1, _NP)
    yW = bp[:, 3].reshape(1, _NP)

    sel = pl.pallas_call(
        _nms_body,
        out_shape=jax.ShapeDtypeStruct((_P, 4), jnp.float32),
        scratch_shapes=[pltpu.VMEM((_NB, 1, _B), jnp.float32)],
    )(bp, x1, y1, x2, y2, xw, yw, xW, yW)

    batch_col = jnp.zeros((_TOPN, 1), jnp.float32)
    return jnp.concatenate([batch_col, sel[:_TOPN]], axis=1)


# speedup vs baseline: 103.8830x; 1.1769x over previous
import functools

import jax
import jax.numpy as jnp
from jax import lax
from jax.experimental import pallas as pl
from jax.experimental.pallas import tpu as pltpu

_TH = 0.7
_N = 5000
_B = 128
_NB = 40
_NP = _NB * _B
_P = 2048
_TOPN = 2000


def _nms_body(
    b_ref, x1_ref, y1_ref, x2_ref, y2_ref, xw_ref, yw_ref, xW_ref, yW_ref,
    out_ref, keep_ref,
):
    f32 = jnp.float32

    keep_ref[:] = jnp.ones((_NB, 1, _B), f32)

    tri = (
        lax.broadcasted_iota(jnp.int32, (_B, _B), 1)
        > lax.broadcasted_iota(jnp.int32, (_B, _B), 0)
    ).astype(f32)
    tx1w_full = xw_ref[:]
    ty1w_full = yw_ref[:]
    tx2w_full = xW_ref[:]
    ty2w_full = yW_ref[:]

    def make_blk_body(c0):
        tx1w = lax.slice(tx1w_full, (0, c0 * _B), (1, _NP))
        ty1w = lax.slice(ty1w_full, (0, c0 * _B), (1, _NP))
        tx2w = lax.slice(tx2w_full, (0, c0 * _B), (1, _NP))
        ty2w = lax.slice(ty2w_full, (0, c0 * _B), (1, _NP))
        taw = (tx2w - tx1w) * (ty2w - ty1w)
        wcol = lax.broadcasted_iota(jnp.int32, (1, _NP - c0 * _B), 1) + c0 * _B

        def blk_body(blk, _):
            pb = b_ref[pl.ds(blk * _B, _B), :]
            px1 = pb[:, 0:1]
            py1 = pb[:, 1:2]
            px2 = pb[:, 2:3]
            py2 = pb[:, 3:4]
            pa = (px2 - px1) * (py2 - py1)

            tx1 = x1_ref[blk]
            ty1 = y1_ref[blk]
            tx2 = x2_ref[blk]
            ty2 = y2_ref[blk]
            ta = (tx2 - tx1) * (ty2 - ty1)
            xx1 = jnp.maximum(px1, tx1)
            yy1 = jnp.maximum(py1, ty1)
            xx2 = jnp.minimum(px2, tx2)
            yy2 = jnp.minimum(py2, ty2)
            w = jnp.maximum(xx2 - xx1, 0.0)
            h = jnp.maximum(yy2 - yy1, 0.0)
            inter = w * h
            iou = inter / (pa + ta - inter + 1e-8)
            m = (iou > _TH).astype(f32) * tri
            elig = keep_ref[blk]

            def fp_cond(carry):
                return carry[1]

            def fp_body(carry):
                k, _ = carry
                sup = lax.dot_general(
                    k, m, (((1,), (0,)), ((), ())), preferred_element_type=f32
                )
                knew = jnp.where(sup > 0.0, 0.0, elig)
                return knew, jnp.any(knew != k)

            k, _ = lax.while_loop(fp_cond, fp_body, (elig, jnp.array(True)))
            keep_ref[blk] = k

            wxx1 = jnp.maximum(px1, tx1w)
            wyy1 = jnp.maximum(py1, ty1w)
            wxx2 = jnp.minimum(px2, tx2w)
            wyy2 = jnp.minimum(py2, ty2w)
            ww = jnp.maximum(wxx2 - wxx1, 0.0)
            wh = jnp.maximum(wyy2 - wyy1, 0.0)
            winter = ww * wh
            wiou = winter / (pa + taw - winter + 1e-8)
            mw = (wiou > _TH).astype(f32)
            sup = lax.dot_general(
                k, mw, (((1,), (0,)), ((), ())), preferred_element_type=f32
            )
            supm = (sup > 0.0) & (wcol >= (blk + 1) * _B)
            for c in range(c0, _NB):
                sc = lax.slice(
                    supm, (0, (c - c0) * _B), (1, (c - c0 + 1) * _B)
                )
                keep_ref[c] = jnp.where(sc, 0.0, keep_ref[c])
            return 0

        return blk_body

    _TIER = 10
    for t0 in range(0, _NB, _TIER):
        lax.fori_loop(t0, t0 + _TIER, make_blk_body(t0), 0)

    keep = keep_ref[:].reshape(_NB, _B)
    gidx = (
        lax.broadcasted_iota(jnp.int32, (_NB, _B), 0) * _B
        + lax.broadcasted_iota(jnp.int32, (_NB, _B), 1)
    )
    validf = (gidx < _N).astype(f32)
    kv = keep * validf
    nv = (1.0 - keep) * validf

    upper = (
        lax.broadcasted_iota(jnp.int32, (_B, _B), 0)
        <= lax.broadcasted_iota(jnp.int32, (_B, _B), 1)
    ).astype(f32)
    dot = functools.partial(
        lax.dot_general,
        dimension_numbers=(((1,), (0,)), ((), ())),
        preferred_element_type=f32,
    )
    kc = dot(kv, upper)
    nc = dot(nv, upper)
    rsk = kc[:, _B - 1 : _B]
    rsn = nc[:, _B - 1 : _B]
    lstrict = (
        lax.broadcasted_iota(jnp.int32, (_NB, _NB), 1)
        < lax.broadcasted_iota(jnp.int32, (_NB, _NB), 0)
    ).astype(f32)
    offk = dot(lstrict, rsk)
    offn = dot(lstrict, rsn)
    nk = jnp.sum(kv)
    posk = kc - 1.0 + offk
    posn = nc - 1.0 + offn + nk
    pos = jnp.where(kv > 0.0, posk, jnp.where(nv > 0.0, posn, 99999.0))

    out_ref[:] = jnp.zeros((_P, 4), f32)
    piota = lax.broadcasted_iota(jnp.int32, (_P, 1), 0).astype(f32)
    for cb in range(_NB):
        prow = lax.slice(pos, (cb, 0), (cb + 1, _B))
        oh = (piota == prow).astype(f32)
        bb = b_ref[cb * _B : (cb + 1) * _B, :]
        out_ref[:] = out_ref[:] + dot(oh, bb)


def kernel(boxes, scores):
    order = jnp.argsort(-scores)
    bs = jnp.take(boxes, order, axis=0)
    bp = jnp.pad(bs, ((0, _NP - _N), (0, 0)))

    x1 = bp[:, 0].reshape(_NB, 1, _B)
    y1 = bp[:, 1].reshape(_NB, 1, _B)
    x2 = bp[:, 2].reshape(_NB, 1, _B)
    y2 = bp[:, 3].reshape(_NB, 1, _B)
    xw = bp[:, 0].reshape(1, _NP)
    yw = bp[:, 1].reshape(1, _NP)
    xW = bp[:, 2].reshape(1, _NP)
    yW = bp[:, 3].reshape(1, _NP)

    sel = pl.pallas_call(
        _nms_body,
        out_shape=jax.ShapeDtypeStruct((_P, 4), jnp.float32),
        scratch_shapes=[pltpu.VMEM((_NB, 1, _B), jnp.float32)],
    )(bp, x1, y1, x2, y2, xw, yw, xW, yW)

    batch_col = jnp.zeros((_TOPN, 1), jnp.float32)
    return jnp.concatenate([batch_col, sel[:_TOPN]], axis=1)
